# baseline (speedup 1.0000x reference)
import functools

import jax
import jax.numpy as jnp
from jax import lax
from jax.experimental import pallas as pl
from jax.experimental.pallas import tpu as pltpu
from jax.experimental.pallas import tpu_sc as plsc

NUM_CORES = 2
NUM_SUBCORES = 16
LANES = 16
NUM_WORKERS = NUM_CORES * NUM_SUBCORES


def _vq_body(chunk, n_groups, x_hbm, p_hbm, o_hbm, buf, par):
    c = lax.axis_index("c")
    s = lax.axis_index("s")
    wid = s * NUM_CORES + c
    base = wid * chunk

    pltpu.sync_copy(x_hbm.at[pl.ds(base, chunk)], buf)
    pltpu.sync_copy(p_hbm, par)

    a0, a1, a2, a3 = par[0], par[1], par[2], par[3]
    b0, b1, b2, b3 = par[4], par[5], par[6], par[7]
    c0, c1, c2, c3 = par[8], par[9], par[10], par[11]

    even0 = lax.iota(jnp.int32, LANES) * 2

    def body(i, _):
        ei = even0 + i * (2 * LANES)
        oi = ei + 1
        xv = plsc.load_gather(buf, [ei])
        yv = plsc.load_gather(buf, [oi])
        s0 = xv * a0 + yv * b0 + c0
        s1 = xv * a1 + yv * b1 + c1
        s2 = xv * a2 + yv * b2 + c2
        s3 = xv * a3 + yv * b3 + c3
        m = s0
        ox = a0
        oy = b0
        g = s1 > m
        m = jnp.maximum(m, s1)
        ox = jnp.where(g, a1, ox)
        oy = jnp.where(g, b1, oy)
        g = s2 > m
        m = jnp.maximum(m, s2)
        ox = jnp.where(g, a2, ox)
        oy = jnp.where(g, b2, oy)
        g = s3 > m
        ox = jnp.where(g, a3, ox)
        oy = jnp.where(g, b3, oy)
        plsc.store_scatter(buf, [ei], ox)
        plsc.store_scatter(buf, [oi], oy)
        return 0

    lax.fori_loop(0, n_groups, body, 0)

    pltpu.sync_copy(buf, o_hbm.at[pl.ds(base, chunk)])


def kernel(x, w, Centroid_sum, Centroid_n):
    n, d = x.shape
    total = n * d
    chunk = total // NUM_WORKERS
    n_groups = chunk // (2 * LANES)

    xflat = jnp.reshape(x, (total,))
    biases = -0.5 * jnp.sum(w * w, axis=0)
    params = jnp.broadcast_to(
        jnp.concatenate([w[0], w[1], biases])[:, None], (12, LANES)
    ).astype(jnp.float32)

    mesh = plsc.VectorSubcoreMesh(
        core_axis_name="c", subcore_axis_name="s",
        num_cores=NUM_CORES, num_subcores=NUM_SUBCORES,
    )
    run = pl.kernel(
        functools.partial(_vq_body, chunk, n_groups),
        out_type=jax.ShapeDtypeStruct((total,), jnp.float32),
        mesh=mesh,
        scratch_types=[
            pltpu.VMEM((chunk,), jnp.float32),
            pltpu.VMEM((12, LANES), jnp.float32),
        ],
        compiler_params=pltpu.CompilerParams(needs_layout_passes=False),
    )
    out = run(xflat, params)
    return jnp.reshape(out, (n, d))


# speedup vs baseline: 56.6650x; 56.6650x over previous
import functools

import jax
import jax.numpy as jnp
from jax import lax
from jax.experimental import pallas as pl
from jax.experimental.pallas import tpu as pltpu
from jax.experimental.pallas import tpu_sc as plsc

NUM_CORES = 2
NUM_SUBCORES = 16
LANES = 16
GROUP = 256
NUM_WORKERS = NUM_CORES * NUM_SUBCORES


def _vq_body(chunk, n_groups, x_hbm, p_hbm, o_hbm, buf, par):
    c = lax.axis_index("c")
    s = lax.axis_index("s")
    wid = s * NUM_CORES + c
    base = wid * chunk

    pltpu.sync_copy(x_hbm.at[pl.ds(base, chunk)], buf)
    pltpu.sync_copy(p_hbm, par)

    a0, a1, a2, a3 = par[0], par[1], par[2], par[3]
    b0, b1, b2, b3 = par[4], par[5], par[6], par[7]
    c0, c1, c2, c3 = par[8], par[9], par[10], par[11]

    def body(g, _):
        goff = g * GROUP
        for u in range(GROUP // (2 * LANES)):
            xo = goff + u * LANES
            yo = xo + (GROUP // 2)
            xv = buf[pl.ds(xo, LANES)]
            yv = buf[pl.ds(yo, LANES)]
            s0 = xv * a0 + yv * b0 + c0
            s1 = xv * a1 + yv * b1 + c1
            s2 = xv * a2 + yv * b2 + c2
            s3 = xv * a3 + yv * b3 + c3
            m = s0
            ox = a0
            oy = b0
            g1 = s1 > m
            m = jnp.maximum(m, s1)
            ox = jnp.where(g1, a1, ox)
            oy = jnp.where(g1, b1, oy)
            g2 = s2 > m
            m = jnp.maximum(m, s2)
            ox = jnp.where(g2, a2, ox)
            oy = jnp.where(g2, b2, oy)
            g3 = s3 > m
            ox = jnp.where(g3, a3, ox)
            oy = jnp.where(g3, b3, oy)
            buf[pl.ds(xo, LANES)] = ox
            buf[pl.ds(yo, LANES)] = oy
        return 0

    lax.fori_loop(0, n_groups, body, 0)

    pltpu.sync_copy(buf, o_hbm.at[pl.ds(base, chunk)])


def kernel(x, w, Centroid_sum, Centroid_n):
    n, d = x.shape
    total = n * d
    chunk = total // NUM_WORKERS
    n_groups = chunk // GROUP

    xt = jnp.transpose(jnp.reshape(x, (n // 128, 128, d)), (0, 2, 1))
    xflat = jnp.reshape(xt, (total,))

    biases = -0.5 * jnp.sum(w * w, axis=0)
    params = jnp.broadcast_to(
        jnp.concatenate([w[0], w[1], biases])[:, None], (12, LANES)
    ).astype(jnp.float32)

    mesh = plsc.VectorSubcoreMesh(
        core_axis_name="c", subcore_axis_name="s",
        num_cores=NUM_CORES, num_subcores=NUM_SUBCORES,
    )
    run = pl.kernel(
        functools.partial(_vq_body, chunk, n_groups),
        out_type=jax.ShapeDtypeStruct((total,), jnp.float32),
        mesh=mesh,
        scratch_types=[
            pltpu.VMEM((chunk,), jnp.float32),
            pltpu.VMEM((12, LANES), jnp.float32),
        ],
        compiler_params=pltpu.CompilerParams(needs_layout_passes=False),
    )
    out = run(xflat, params)
    out3 = jnp.reshape(out, (n // 128, d, 128))
    return jnp.reshape(jnp.transpose(out3, (0, 2, 1)), (n, d))
